# baseline (speedup 1.0000x reference)
import functools

import jax
import jax.numpy as jnp
from jax import lax
from jax.experimental import pallas as pl
from jax.experimental.pallas import tpu as pltpu
from jax.experimental.pallas import tpu_sc as plsc

KNN_EXTRA = 100
MASK_DISTANCE = 1000.0
KNN_OUT = 10
TOTAL_FETCH = KNN_OUT + KNN_EXTRA + 1

NQ = 1024
N_KEYS = 100000
N_PAD = 100352
NB = 2048
NV = N_PAD // 16
NCH = N_PAD // 128
QB = 256
CAP = 256
CHCAP = 512
GB = 128
RPW = 32


def _dist_body(q_ref, k_ref, qs_ref, ks_ref, d_ref, m16_ref):
    q = q_ref[...]
    k = k_ref[...]
    qk = lax.dot_general(q, k, (((1,), (1,)), ((), ())),
                         preferred_element_type=jnp.float32)
    d = qs_ref[...] - 2.0 * qk + ks_ref[...][0:1, :]
    j = pl.program_id(1)
    cols = j * NB + lax.broadcasted_iota(jnp.int32, (QB, NB), 1)
    d = jnp.where(cols < N_KEYS, d, jnp.float32(3e38))
    d_ref[...] = d.reshape(QB, NB // 128, 128)
    m = jnp.min(d.reshape(QB, NB // 128, 128), axis=-1)
    mrep = jnp.broadcast_to(m[:, :, None], (QB, NB // 128, 8))
    m16_ref[...] = mrep.reshape(QB, NB // 16)


def _dist_call(queries, keys_pad, q_sq, ks8):
    return pl.pallas_call(
        _dist_body,
        grid=(NQ // QB, N_PAD // NB),
        in_specs=[
            pl.BlockSpec((QB, 128), lambda i, j: (i, 0)),
            pl.BlockSpec((NB, 128), lambda i, j: (j, 0)),
            pl.BlockSpec((QB, 1), lambda i, j: (i, 0)),
            pl.BlockSpec((8, NB), lambda i, j: (0, j)),
        ],
        out_specs=[
            pl.BlockSpec((QB, NB // 128, 128), lambda i, j: (i, j, 0)),
            pl.BlockSpec((QB, NB // 16), lambda i, j: (i, j)),
        ],
        out_shape=[
            jax.ShapeDtypeStruct((NQ, NCH, 128), jnp.float32),
            jax.ShapeDtypeStruct((NQ, NV), jnp.float32),
        ],
        compiler_params=pltpu.CompilerParams(
            dimension_semantics=("arbitrary", "arbitrary"),
        ),
    )(queries, keys_pad, q_sq, ks8)


def _tau_body(mrep_ref, tau_ref):
    x = mrep_ref[...]
    lane = lax.broadcasted_iota(jnp.int32, (QB, NV), 1)
    rep_mask = (lane & 7) == 0
    inf = jnp.float32(jnp.inf)
    hm = jnp.where(rep_mask, x, inf)
    lo = jnp.min(x, axis=1, keepdims=True)
    real_mask = rep_mask & ((lane >> 3) <= (N_KEYS - 1) // 128)
    hi = jnp.max(jnp.where(real_mask, x, -inf), axis=1, keepdims=True)

    def count_le(t):
        return jnp.sum((hm <= t).astype(jnp.int32), axis=1, keepdims=True)

    def body(it, carry):
        lo, hi = carry
        mid = 0.5 * (lo + hi)
        ge = count_le(mid) >= TOTAL_FETCH
        hi = jnp.where(ge, mid, hi)
        lo = jnp.where(ge, lo, mid)
        return lo, hi

    lo, hi = lax.fori_loop(0, 26, body, (lo, hi))
    tau = jnp.where(count_le(lo) >= TOTAL_FETCH, lo, hi)
    tau_ref[...] = jnp.broadcast_to(tau, (QB, 128))


def _tau_call(mrep):
    return pl.pallas_call(
        _tau_body,
        grid=(NQ // QB,),
        in_specs=[pl.BlockSpec((QB, NV), lambda i: (i, 0))],
        out_specs=pl.BlockSpec((QB, 128), lambda i: (i, 0)),
        out_shape=jax.ShapeDtypeStruct((NQ, 128), jnp.float32),
    )(mrep)


def _sc_select_call(dist2dc, mrepf, tauf, doc_ids):
    mesh = plsc.VectorSubcoreMesh(core_axis_name="c", subcore_axis_name="s")
    nc = 2
    inf = jnp.float32(jnp.inf)

    @functools.partial(
        pl.kernel,
        out_type=(
            jax.ShapeDtypeStruct((NQ * CAP,), jnp.float32),
            jax.ShapeDtypeStruct((NQ * CAP,), jnp.int32),
            jax.ShapeDtypeStruct((NQ * CAP,), jnp.int32),
        ),
        mesh=mesh,
        scratch_types=[
            pltpu.VMEM((NV,), jnp.float32),
            pltpu.VMEM((RPW * 128,), jnp.float32),
            pltpu.VMEM((CHCAP,), jnp.int32),
            pltpu.VMEM((GB, 128), jnp.float32),
            pltpu.VMEM((CAP,), jnp.float32),
            pltpu.VMEM((CAP,), jnp.int32),
            pltpu.VMEM((CAP,), jnp.int32),
            pltpu.SemaphoreType.DMA,
        ],
        compiler_params=pltpu.CompilerParams(needs_layout_passes=False),
    )
    def sc_select(dist2dc, mrepf, tauf, doc_ids,
                  valsf, idxf, docf,
                  mrep_v, tau_v, cid_v, gbuf, cand_v, cand_i, cand_d, sem):
        wid = lax.axis_index("s") * nc + lax.axis_index("c")
        row0 = wid * RPW
        iota = lax.broadcasted_iota(jnp.int32, (16,), 0)
        pltpu.sync_copy(tauf.at[pl.ds(row0 * 128, RPW * 128)], tau_v)
        for z in range(CHCAP // 16):
            cid_v[pl.ds(z * 16, 16)] = iota + z * 16

        rep_mask = (iota & 7) == 0

        def row_body(r_local, _):
            row = row0 + r_local
            pltpu.sync_copy(mrepf.at[pl.ds(row * NV, NV)], mrep_v)
            tau_vec = tau_v[pl.ds(r_local * 128, 16)]

            def scan_body(g, cnt):
                m = mrep_v[pl.ds(g * 16, 16)]
                msk = jnp.logical_and(m <= tau_vec, rep_mask)
                s = jnp.sum(msk.astype(jnp.int32))

                @pl.when(s > 0)
                def _():
                    key = jnp.where(msk, m, inf)
                    ids = row * NCH + g * 2 + (iota >> 3)
                    sk, sv = plsc.sort_key_val(key, ids)
                    pos = jnp.minimum(cnt, CHCAP - 16)
                    cid_v[pl.ds(pos, 16)] = sv

                return jnp.minimum(cnt + s, CHCAP - 16)

            cnt = lax.fori_loop(0, NV // 16, scan_body, jnp.int32(0))

            for z in range(CAP // 16):
                cand_v[pl.ds(z * 16, 16)] = jnp.full((16,), inf, jnp.float32)
                cand_i[pl.ds(z * 16, 16)] = iota + z * 16

            nb = (cnt + (GB - 1)) // GB

            def batch_body(b, ocnt):
                pltpu.async_copy(
                    dist2dc.at[cid_v.at[pl.ds(b * GB, GB)]], gbuf, sem
                ).wait()

                def grp_body(sg, ocnt):
                    cids16 = cid_v[pl.ds(b * GB + sg * 16, 16)]
                    for u in range(16):
                        gslot = sg * 16 + u
                        kb = (cids16[u] - row * NCH) * 128
                        valid = b * GB + gslot < cnt
                        for vv in range(8):
                            v = gbuf[gslot, pl.ds(vv * 16, 16)]
                            kidx = jnp.broadcast_to(kb + vv * 16, (16,)) + iota
                            keep = jnp.logical_and(
                                v <= tau_vec,
                                jnp.logical_and(valid, ocnt <= CAP - 16))
                            s = jnp.sum(keep.astype(jnp.int32))

                            @pl.when(s > 0)
                            def _():
                                key = jnp.where(keep, v, inf)
                                sk, sv = plsc.sort_key_val(
                                    key, jnp.where(keep, kidx, iota))
                                pos = jnp.minimum(ocnt, CAP - 16)
                                cand_v[pl.ds(pos, 16)] = sk
                                cand_i[pl.ds(pos, 16)] = sv

                            ocnt = ocnt + s
                    return ocnt

                return lax.fori_loop(0, GB // 16, grp_body, ocnt)

            lax.fori_loop(0, nb, batch_body, jnp.int32(0))

            pltpu.async_copy(doc_ids.at[cand_i.at[pl.ds(0, 128)]],
                             cand_d.at[pl.ds(0, 128)], sem).wait()
            pltpu.async_copy(doc_ids.at[cand_i.at[pl.ds(128, 128)]],
                             cand_d.at[pl.ds(128, 128)], sem).wait()

            pltpu.sync_copy(cand_v, valsf.at[pl.ds(row * CAP, CAP)])
            pltpu.sync_copy(cand_i, idxf.at[pl.ds(row * CAP, CAP)])
            pltpu.sync_copy(cand_d, docf.at[pl.ds(row * CAP, CAP)])
            return 0

        lax.fori_loop(0, RPW, row_body, 0)

    return sc_select(dist2dc, mrepf, tauf, doc_ids)


def _lexlt(a_v, a_i, b_v, b_i):
    return (a_v < b_v) | ((a_v == b_v) & (a_i < b_i))


def _bitonic(arrs, width, key2):
    lane = lax.broadcasted_iota(jnp.int32, (QB, width), 1)
    k = 2
    while k <= width:
        j = k // 2
        while j >= 1:
            low = (lane & j) == 0
            partners = [jnp.where(low, jnp.roll(a, -j, axis=1),
                                  jnp.roll(a, j, axis=1)) for a in arrs]
            keep_min = low == ((lane & k) == 0)
            lt = _lexlt(arrs[0], arrs[1], partners[0], partners[1]) if key2 \
                else (arrs[0] < partners[0])
            take_self = lt == keep_min
            arrs = [jnp.where(take_self, a, p) for a, p in zip(arrs, partners)]
            j //= 2
        k *= 2
    return arrs


def _final_body(v_ref, i_ref, d_ref, qd_ref, od_ref, oi_ref):
    v = v_ref[...]
    i = i_ref[...]
    d = d_ref[...]
    qd = qd_ref[...]
    i_f = i.astype(jnp.float32)
    v, i_f, d_f = _bitonic((v, i_f, d.astype(jnp.float32)), CAP, key2=True)
    pos = lax.broadcasted_iota(jnp.int32, (QB, CAP), 1)
    alive = pos < TOTAL_FETCH
    same = d_f == jnp.broadcast_to(qd.astype(jnp.float32), (QB, CAP))
    mv = jnp.where(alive & same, jnp.float32(MASK_DISTANCE),
                   jnp.where(alive, v, jnp.float32(jnp.inf)))
    rank = pos.astype(jnp.float32)
    mv, rank, i_f = _bitonic((mv, rank, i_f), CAP, key2=True)
    od_ref[...] = mv[:, :KNN_OUT]
    oi_ref[...] = i_f[:, :KNN_OUT].astype(jnp.int32)


def _final_call(vals, idx, doc, qdoc):
    return pl.pallas_call(
        _final_body,
        grid=(NQ // QB,),
        in_specs=[
            pl.BlockSpec((QB, CAP), lambda i: (i, 0)),
            pl.BlockSpec((QB, CAP), lambda i: (i, 0)),
            pl.BlockSpec((QB, CAP), lambda i: (i, 0)),
            pl.BlockSpec((QB, 1), lambda i: (i, 0)),
        ],
        out_specs=[
            pl.BlockSpec((QB, KNN_OUT), lambda i: (i, 0)),
            pl.BlockSpec((QB, KNN_OUT), lambda i: (i, 0)),
        ],
        out_shape=[
            jax.ShapeDtypeStruct((NQ, KNN_OUT), jnp.float32),
            jax.ShapeDtypeStruct((NQ, KNN_OUT), jnp.int32),
        ],
    )(vals, idx, doc, qdoc)


def kernel(queries, keys, doc_ids, query_doc_ids, knn):
    keys_pad = jnp.pad(keys, ((0, N_PAD - N_KEYS), (0, 0)))
    q_sq = jnp.sum(queries * queries, axis=1, keepdims=True)
    k_sq = jnp.sum(keys_pad * keys_pad, axis=1)
    ks8 = jnp.broadcast_to(k_sq[None, :], (8, N_PAD))
    dist, mrep = _dist_call(queries, keys_pad, q_sq, ks8)
    tau = _tau_call(mrep)
    valsf, idxf, docf = _sc_select_call(
        dist.reshape(NQ * NCH, 128),
        mrep.reshape(NQ * NV),
        tau.reshape(NQ * 128),
        doc_ids,
    )
    dists_out, idx_out = _final_call(
        valsf.reshape(NQ, CAP),
        idxf.reshape(NQ, CAP),
        docf.reshape(NQ, CAP),
        query_doc_ids.reshape(NQ, 1),
    )
    return dists_out, idx_out


# speedup vs baseline: 1.2232x; 1.2232x over previous
import functools

import jax
import jax.numpy as jnp
from jax import lax
from jax.experimental import pallas as pl
from jax.experimental.pallas import tpu as pltpu
from jax.experimental.pallas import tpu_sc as plsc

KNN_EXTRA = 100
MASK_DISTANCE = 1000.0
KNN_OUT = 10
TOTAL_FETCH = KNN_OUT + KNN_EXTRA + 1

NQ = 1024
N_KEYS = 100000
N_PAD = 100352
NB = 2048
NV = N_PAD // 16
NCH = N_PAD // 128
QB = 256
CAP = 256
CHCAP = 512
GB = 128
RPW = 32


def _dist_body(q_ref, k_ref, qs_ref, ks_ref, d_ref, m16_ref):
    q = q_ref[...]
    k = k_ref[...]
    qk = lax.dot_general(q, k, (((1,), (1,)), ((), ())),
                         preferred_element_type=jnp.float32)
    d = qs_ref[...] - 2.0 * qk + ks_ref[...][0:1, :]
    j = pl.program_id(1)
    cols = j * NB + lax.broadcasted_iota(jnp.int32, (QB, NB), 1)
    d = jnp.where(cols < N_KEYS, d, jnp.float32(3e38))
    d_ref[...] = d.reshape(QB, NB // 128, 128)
    m = jnp.min(d.reshape(QB, NB // 128, 128), axis=-1)
    mrep = jnp.broadcast_to(m[:, :, None], (QB, NB // 128, 8))
    m16_ref[...] = mrep.reshape(QB, NB // 16)


def _dist_call(queries, keys_pad, q_sq, ks8):
    return pl.pallas_call(
        _dist_body,
        grid=(NQ // QB, N_PAD // NB),
        in_specs=[
            pl.BlockSpec((QB, 128), lambda i, j: (i, 0)),
            pl.BlockSpec((NB, 128), lambda i, j: (j, 0)),
            pl.BlockSpec((QB, 1), lambda i, j: (i, 0)),
            pl.BlockSpec((8, NB), lambda i, j: (0, j)),
        ],
        out_specs=[
            pl.BlockSpec((QB, NB // 128, 128), lambda i, j: (i, j, 0)),
            pl.BlockSpec((QB, NB // 16), lambda i, j: (i, j)),
        ],
        out_shape=[
            jax.ShapeDtypeStruct((NQ, NCH, 128), jnp.float32),
            jax.ShapeDtypeStruct((NQ, NV), jnp.float32),
        ],
        compiler_params=pltpu.CompilerParams(
            dimension_semantics=("arbitrary", "arbitrary"),
        ),
    )(queries, keys_pad, q_sq, ks8)


def _tau_body(mrep_ref, tau_ref):
    x = mrep_ref[...]
    lane = lax.broadcasted_iota(jnp.int32, (QB, NV), 1)
    rep_mask = (lane & 7) == 0
    inf = jnp.float32(jnp.inf)
    hm = jnp.where(rep_mask, x, inf)
    lo = jnp.min(x, axis=1, keepdims=True)
    real_mask = rep_mask & ((lane >> 3) <= (N_KEYS - 1) // 128)
    hi = jnp.max(jnp.where(real_mask, x, -inf), axis=1, keepdims=True)

    def count_le(t):
        return jnp.sum((hm <= t).astype(jnp.int32), axis=1, keepdims=True)

    def body(it, carry):
        lo, hi = carry
        mid = 0.5 * (lo + hi)
        ge = count_le(mid) >= TOTAL_FETCH
        hi = jnp.where(ge, mid, hi)
        lo = jnp.where(ge, lo, mid)
        return lo, hi

    lo, hi = lax.fori_loop(0, 26, body, (lo, hi))
    tau = jnp.where(count_le(lo) >= TOTAL_FETCH, lo, hi)
    tau_ref[...] = jnp.broadcast_to(tau, (QB, 128))


def _tau_call(mrep):
    return pl.pallas_call(
        _tau_body,
        grid=(NQ // QB,),
        in_specs=[pl.BlockSpec((QB, NV), lambda i: (i, 0))],
        out_specs=pl.BlockSpec((QB, 128), lambda i: (i, 0)),
        out_shape=jax.ShapeDtypeStruct((NQ, 128), jnp.float32),
    )(mrep)


def _sc_select_call(dist2dc, mrepf, tauf, doc_ids):
    mesh = plsc.VectorSubcoreMesh(core_axis_name="c", subcore_axis_name="s")
    nc = 2
    inf = jnp.float32(jnp.inf)

    @functools.partial(
        pl.kernel,
        out_type=(
            jax.ShapeDtypeStruct((NQ * CAP,), jnp.float32),
            jax.ShapeDtypeStruct((NQ * CAP,), jnp.int32),
            jax.ShapeDtypeStruct((NQ * CAP,), jnp.int32),
        ),
        mesh=mesh,
        scratch_types=[
            pltpu.VMEM((NV,), jnp.float32),
            pltpu.VMEM((RPW * 128,), jnp.float32),
            pltpu.VMEM((CHCAP,), jnp.int32),
            pltpu.VMEM((GB, 128), jnp.float32),
            pltpu.VMEM((CAP,), jnp.float32),
            pltpu.VMEM((CAP,), jnp.int32),
            pltpu.VMEM((CAP,), jnp.int32),
            pltpu.SemaphoreType.DMA,
        ],
        compiler_params=pltpu.CompilerParams(needs_layout_passes=False),
    )
    def sc_select(dist2dc, mrepf, tauf, doc_ids,
                  valsf, idxf, docf,
                  mrep_v, tau_v, cid_v, gbuf, cand_v, cand_i, cand_d, sem):
        wid = lax.axis_index("s") * nc + lax.axis_index("c")
        row0 = wid * RPW
        iota = lax.broadcasted_iota(jnp.int32, (16,), 0)
        pltpu.sync_copy(tauf.at[pl.ds(row0 * 128, RPW * 128)], tau_v)
        for z in range(CHCAP // 16):
            cid_v[pl.ds(z * 16, 16)] = iota + z * 16

        rep_mask = (iota & 7) == 0

        def row_body(r_local, _):
            row = row0 + r_local
            pltpu.sync_copy(mrepf.at[pl.ds(row * NV, NV)], mrep_v)
            tau_vec = tau_v[pl.ds(r_local * 128, 16)]

            def scan_body(g, cnt):
                m = mrep_v[pl.ds(g * 16, 16)]
                msk = jnp.logical_and(m <= tau_vec, rep_mask)
                key = jnp.where(msk, m, inf)
                ids = row * NCH + g * 2 + (iota >> 3)
                sk, sv = plsc.sort_key_val(key, ids)
                pos = jnp.minimum(cnt, CHCAP - 16)
                cid_v[pl.ds(pos, 16)] = sv
                npos = cnt + jnp.sum(msk.astype(jnp.int32))
                return jnp.minimum(npos, CHCAP - 16)

            cnt = lax.fori_loop(0, NV // 16, scan_body, jnp.int32(0))

            for z in range(CAP // 16):
                cand_v[pl.ds(z * 16, 16)] = jnp.full((16,), inf, jnp.float32)
                cand_i[pl.ds(z * 16, 16)] = iota + z * 16

            nb = (cnt + (GB - 1)) // GB

            def batch_body(b, ocnt):
                pltpu.async_copy(
                    dist2dc.at[cid_v.at[pl.ds(b * GB, GB)]], gbuf, sem
                ).wait()

                def grp_body(sg, ocnt):
                    cids16 = cid_v[pl.ds(b * GB + sg * 16, 16)]
                    for u in range(16):
                        gslot = sg * 16 + u
                        kb = (cids16[u] - row * NCH) * 128
                        valid = b * GB + gslot < cnt
                        for vv in range(8):
                            v = gbuf[gslot, pl.ds(vv * 16, 16)]
                            kidx = jnp.broadcast_to(kb + vv * 16, (16,)) + iota
                            keep = jnp.logical_and(
                                v <= tau_vec,
                                jnp.logical_and(valid, ocnt <= CAP - 16))
                            key = jnp.where(keep, v, inf)
                            sk, sv = plsc.sort_key_val(
                                key, jnp.where(keep, kidx, iota))
                            pos = jnp.minimum(ocnt, CAP - 16)
                            cand_v[pl.ds(pos, 16)] = sk
                            cand_i[pl.ds(pos, 16)] = sv
                            ocnt = ocnt + jnp.sum(keep.astype(jnp.int32))
                    return ocnt

                return lax.fori_loop(0, GB // 16, grp_body, ocnt)

            lax.fori_loop(0, nb, batch_body, jnp.int32(0))

            pltpu.async_copy(doc_ids.at[cand_i.at[pl.ds(0, 128)]],
                             cand_d.at[pl.ds(0, 128)], sem).wait()
            pltpu.async_copy(doc_ids.at[cand_i.at[pl.ds(128, 128)]],
                             cand_d.at[pl.ds(128, 128)], sem).wait()

            pltpu.sync_copy(cand_v, valsf.at[pl.ds(row * CAP, CAP)])
            pltpu.sync_copy(cand_i, idxf.at[pl.ds(row * CAP, CAP)])
            pltpu.sync_copy(cand_d, docf.at[pl.ds(row * CAP, CAP)])
            return 0

        lax.fori_loop(0, RPW, row_body, 0)

    return sc_select(dist2dc, mrepf, tauf, doc_ids)


def _lexlt(a_v, a_i, b_v, b_i):
    return (a_v < b_v) | ((a_v == b_v) & (a_i < b_i))


def _bitonic(arrs, width, key2):
    lane = lax.broadcasted_iota(jnp.int32, (QB, width), 1)
    k = 2
    while k <= width:
        j = k // 2
        while j >= 1:
            low = (lane & j) == 0
            partners = [jnp.where(low, jnp.roll(a, -j, axis=1),
                                  jnp.roll(a, j, axis=1)) for a in arrs]
            keep_min = low == ((lane & k) == 0)
            lt = _lexlt(arrs[0], arrs[1], partners[0], partners[1]) if key2 \
                else (arrs[0] < partners[0])
            take_self = lt == keep_min
            arrs = [jnp.where(take_self, a, p) for a, p in zip(arrs, partners)]
            j //= 2
        k *= 2
    return arrs


def _final_body(v_ref, i_ref, d_ref, qd_ref, od_ref, oi_ref):
    v = v_ref[...]
    i = i_ref[...]
    d = d_ref[...]
    qd = qd_ref[...]
    i_f = i.astype(jnp.float32)
    v, i_f, d_f = _bitonic((v, i_f, d.astype(jnp.float32)), CAP, key2=True)
    pos = lax.broadcasted_iota(jnp.int32, (QB, CAP), 1)
    alive = pos < TOTAL_FETCH
    same = d_f == jnp.broadcast_to(qd.astype(jnp.float32), (QB, CAP))
    mv = jnp.where(alive & same, jnp.float32(MASK_DISTANCE),
                   jnp.where(alive, v, jnp.float32(jnp.inf)))
    rank = pos.astype(jnp.float32)
    mv, rank, i_f = _bitonic((mv, rank, i_f), CAP, key2=True)
    od_ref[...] = mv[:, :KNN_OUT]
    oi_ref[...] = i_f[:, :KNN_OUT].astype(jnp.int32)


def _final_call(vals, idx, doc, qdoc):
    return pl.pallas_call(
        _final_body,
        grid=(NQ // QB,),
        in_specs=[
            pl.BlockSpec((QB, CAP), lambda i: (i, 0)),
            pl.BlockSpec((QB, CAP), lambda i: (i, 0)),
            pl.BlockSpec((QB, CAP), lambda i: (i, 0)),
            pl.BlockSpec((QB, 1), lambda i: (i, 0)),
        ],
        out_specs=[
            pl.BlockSpec((QB, KNN_OUT), lambda i: (i, 0)),
            pl.BlockSpec((QB, KNN_OUT), lambda i: (i, 0)),
        ],
        out_shape=[
            jax.ShapeDtypeStruct((NQ, KNN_OUT), jnp.float32),
            jax.ShapeDtypeStruct((NQ, KNN_OUT), jnp.int32),
        ],
    )(vals, idx, doc, qdoc)


def kernel(queries, keys, doc_ids, query_doc_ids, knn):
    keys_pad = jnp.pad(keys, ((0, N_PAD - N_KEYS), (0, 0)))
    q_sq = jnp.sum(queries * queries, axis=1, keepdims=True)
    k_sq = jnp.sum(keys_pad * keys_pad, axis=1)
    ks8 = jnp.broadcast_to(k_sq[None, :], (8, N_PAD))
    dist, mrep = _dist_call(queries, keys_pad, q_sq, ks8)
    tau = _tau_call(mrep)
    valsf, idxf, docf = _sc_select_call(
        dist.reshape(NQ * NCH, 128),
        mrep.reshape(NQ * NV),
        tau.reshape(NQ * 128),
        doc_ids,
    )
    dists_out, idx_out = _final_call(
        valsf.reshape(NQ, CAP),
        idxf.reshape(NQ, CAP),
        docf.reshape(NQ, CAP),
        query_doc_ids.reshape(NQ, 1),
    )
    return dists_out, idx_out


# speedup vs baseline: 1.2835x; 1.0493x over previous
import functools

import jax
import jax.numpy as jnp
from jax import lax
from jax.experimental import pallas as pl
from jax.experimental.pallas import tpu as pltpu
from jax.experimental.pallas import tpu_sc as plsc

KNN_EXTRA = 100
MASK_DISTANCE = 1000.0
KNN_OUT = 10
TOTAL_FETCH = KNN_OUT + KNN_EXTRA + 1

NQ = 1024
N_KEYS = 100000
N_PAD = 100352
NB = 2048
NV = N_PAD // 16
NCH = N_PAD // 128
QB = 256
CAP = 256
CHCAP = 512
GB = 128
RPW = 32


def _dist_body(q_ref, k_ref, qs_ref, ks_ref, d_ref, m16_ref):
    q = q_ref[...]
    k = k_ref[...]
    qk = lax.dot_general(q, k, (((1,), (1,)), ((), ())),
                         preferred_element_type=jnp.float32)
    d = qs_ref[...] - 2.0 * qk + ks_ref[...][0:1, :]
    j = pl.program_id(1)
    cols = j * NB + lax.broadcasted_iota(jnp.int32, (QB, NB), 1)
    d = jnp.where(cols < N_KEYS, d, jnp.float32(3e38))
    d_ref[...] = d.reshape(QB, NB // 128, 128)
    m = jnp.min(d.reshape(QB, NB // 128, 128), axis=-1)
    mrep = jnp.broadcast_to(m[:, :, None], (QB, NB // 128, 8))
    m16_ref[...] = mrep.reshape(QB, NB // 16)


def _dist_call(queries, keys_pad, q_sq, ks8):
    return pl.pallas_call(
        _dist_body,
        grid=(NQ // QB, N_PAD // NB),
        in_specs=[
            pl.BlockSpec((QB, 128), lambda i, j: (i, 0)),
            pl.BlockSpec((NB, 128), lambda i, j: (j, 0)),
            pl.BlockSpec((QB, 1), lambda i, j: (i, 0)),
            pl.BlockSpec((8, NB), lambda i, j: (0, j)),
        ],
        out_specs=[
            pl.BlockSpec((QB, NB // 128, 128), lambda i, j: (i, j, 0)),
            pl.BlockSpec((QB, NB // 16), lambda i, j: (i, j)),
        ],
        out_shape=[
            jax.ShapeDtypeStruct((NQ, NCH, 128), jnp.float32),
            jax.ShapeDtypeStruct((NQ, NV), jnp.float32),
        ],
        compiler_params=pltpu.CompilerParams(
            dimension_semantics=("arbitrary", "arbitrary"),
        ),
    )(queries, keys_pad, q_sq, ks8)


def _tau_body(mrep_ref, tau_ref):
    x = mrep_ref[...]
    lane = lax.broadcasted_iota(jnp.int32, (QB, NV), 1)
    rep_mask = (lane & 7) == 0
    inf = jnp.float32(jnp.inf)
    hm = jnp.where(rep_mask, x, inf)
    lo = jnp.min(x, axis=1, keepdims=True)
    real_mask = rep_mask & ((lane >> 3) <= (N_KEYS - 1) // 128)
    hi = jnp.max(jnp.where(real_mask, x, -inf), axis=1, keepdims=True)

    def count_le(t):
        return jnp.sum((hm <= t).astype(jnp.int32), axis=1, keepdims=True)

    def body(it, carry):
        lo, hi = carry
        mid = 0.5 * (lo + hi)
        ge = count_le(mid) >= TOTAL_FETCH
        hi = jnp.where(ge, mid, hi)
        lo = jnp.where(ge, lo, mid)
        return lo, hi

    lo, hi = lax.fori_loop(0, 26, body, (lo, hi))
    tau = jnp.where(count_le(lo) >= TOTAL_FETCH, lo, hi)
    tau_ref[...] = jnp.broadcast_to(tau, (QB, 128))


def _tau_call(mrep):
    return pl.pallas_call(
        _tau_body,
        grid=(NQ // QB,),
        in_specs=[pl.BlockSpec((QB, NV), lambda i: (i, 0))],
        out_specs=pl.BlockSpec((QB, 128), lambda i: (i, 0)),
        out_shape=jax.ShapeDtypeStruct((NQ, 128), jnp.float32),
    )(mrep)


def _sc_select_call(dist2dc, mrepf, tauf, doc_ids):
    mesh = plsc.VectorSubcoreMesh(core_axis_name="c", subcore_axis_name="s")
    nc = 2
    inf = jnp.float32(jnp.inf)

    @functools.partial(
        pl.kernel,
        out_type=(
            jax.ShapeDtypeStruct((NQ * CAP,), jnp.float32),
            jax.ShapeDtypeStruct((NQ * CAP,), jnp.int32),
            jax.ShapeDtypeStruct((NQ * CAP,), jnp.int32),
        ),
        mesh=mesh,
        scratch_types=[
            pltpu.VMEM((NV,), jnp.float32),
            pltpu.VMEM((RPW * 128,), jnp.float32),
            pltpu.VMEM((CHCAP,), jnp.int32),
            pltpu.VMEM((GB, 128), jnp.float32),
            pltpu.VMEM((CAP,), jnp.float32),
            pltpu.VMEM((CAP,), jnp.int32),
            pltpu.VMEM((CAP,), jnp.int32),
            pltpu.SemaphoreType.DMA,
        ],
        compiler_params=pltpu.CompilerParams(needs_layout_passes=False),
    )
    def sc_select(dist2dc, mrepf, tauf, doc_ids,
                  valsf, idxf, docf,
                  mrep_v, tau_v, cid_v, gbuf, cand_v, cand_i, cand_d, sem):
        wid = lax.axis_index("s") * nc + lax.axis_index("c")
        row0 = wid * RPW
        iota = lax.broadcasted_iota(jnp.int32, (16,), 0)
        pltpu.sync_copy(tauf.at[pl.ds(row0 * 128, RPW * 128)], tau_v)
        for z in range(CHCAP // 16):
            cid_v[pl.ds(z * 16, 16)] = iota + z * 16

        rep_mask = (iota & 7) == 0

        def row_body(r_local, _):
            row = row0 + r_local
            pltpu.sync_copy(mrepf.at[pl.ds(row * NV, NV)], mrep_v)
            tau_vec = tau_v[pl.ds(r_local * 128, 16)]

            def scan_body(g, cnt):
                m = mrep_v[pl.ds(g * 16, 16)]
                msk = jnp.logical_and(m <= tau_vec, rep_mask)
                key = jnp.where(msk, m, inf)
                ids = row * NCH + g * 2 + (iota >> 3)
                sk, sv = plsc.sort_key_val(key, ids)
                pos = jnp.minimum(cnt, CHCAP - 16)
                cid_v[pl.ds(pos, 16)] = sv
                npos = cnt + jnp.sum(msk.astype(jnp.int32))
                return jnp.minimum(npos, CHCAP - 16)

            cnt = lax.fori_loop(0, NV // 16, scan_body, jnp.int32(0))

            for z in range(CAP // 16):
                cand_v[pl.ds(z * 16, 16)] = jnp.full((16,), inf, jnp.float32)
                cand_i[pl.ds(z * 16, 16)] = iota + z * 16

            nb = (cnt + (GB - 1)) // GB

            def batch_body(b, ocnt):
                pltpu.async_copy(
                    dist2dc.at[cid_v.at[pl.ds(b * GB, GB)]], gbuf, sem
                ).wait()

                def grp_body(sg, ocnt):
                    cids16 = cid_v[pl.ds(b * GB + sg * 16, 16)]
                    for u in range(16):
                        gslot = sg * 16 + u
                        kb = (cids16[u] - row * NCH) * 128
                        valid = b * GB + gslot < cnt
                        for vv in range(8):
                            v = gbuf[gslot, pl.ds(vv * 16, 16)]
                            kidx = jnp.broadcast_to(kb + vv * 16, (16,)) + iota
                            keep = jnp.logical_and(
                                v <= tau_vec,
                                jnp.logical_and(valid, ocnt <= CAP - 16))
                            key = jnp.where(keep, v, inf)
                            sk, sv = plsc.sort_key_val(
                                key, jnp.where(keep, kidx, iota))
                            pos = jnp.minimum(ocnt, CAP - 16)
                            cand_v[pl.ds(pos, 16)] = sk
                            cand_i[pl.ds(pos, 16)] = sv
                            ocnt = ocnt + jnp.sum(keep.astype(jnp.int32))
                    return ocnt

                return lax.fori_loop(0, GB // 16, grp_body, ocnt)

            lax.fori_loop(0, nb, batch_body, jnp.int32(0))

            pltpu.async_copy(doc_ids.at[cand_i.at[pl.ds(0, 128)]],
                             cand_d.at[pl.ds(0, 128)], sem).wait()
            pltpu.async_copy(doc_ids.at[cand_i.at[pl.ds(128, 128)]],
                             cand_d.at[pl.ds(128, 128)], sem).wait()

            pltpu.sync_copy(cand_v, valsf.at[pl.ds(row * CAP, CAP)])
            pltpu.sync_copy(cand_i, idxf.at[pl.ds(row * CAP, CAP)])
            pltpu.sync_copy(cand_d, docf.at[pl.ds(row * CAP, CAP)])
            return 0

        lax.fori_loop(0, RPW, row_body, 0)

    return sc_select(dist2dc, mrepf, tauf, doc_ids)


def _lexlt(a_v, a_i, b_v, b_i):
    return (a_v < b_v) | ((a_v == b_v) & (a_i < b_i))


def _bitonic(arrs, width, key2):
    arrs = list(arrs)
    lane = lax.broadcasted_iota(jnp.int32, (QB, width), 1)
    k = 2
    while k <= width:
        j = k // 2
        while j >= 1:
            low = (lane & j) == 0
            partners = [jnp.where(low, jnp.roll(a, -j, axis=1),
                                  jnp.roll(a, j, axis=1)) for a in arrs]
            keep_min = low == ((lane & k) == 0)
            lt = _lexlt(arrs[0], arrs[1], partners[0], partners[1]) if key2 \
                else (arrs[0] < partners[0])
            take_self = lt == keep_min
            arrs = [jnp.where(take_self, a, p) for a, p in zip(arrs, partners)]
            j //= 2
        k *= 2
    return arrs


def _final_body(v_ref, i_ref, d_ref, qd_ref, od_ref, oi_ref):
    v = v_ref[...]
    i = i_ref[...]
    d = d_ref[...]
    qd = qd_ref[...]
    i_f = i.astype(jnp.float32)
    v, i_f, d_f = _bitonic((v, i_f, d.astype(jnp.float32)), CAP, key2=True)
    W2 = 128
    v, i_f, d_f = v[:, :W2], i_f[:, :W2], d_f[:, :W2]
    pos = lax.broadcasted_iota(jnp.int32, (QB, W2), 1)
    alive = pos < TOTAL_FETCH
    same = d_f == jnp.broadcast_to(qd.astype(jnp.float32), (QB, W2))
    mv = jnp.where(alive & same, jnp.float32(MASK_DISTANCE),
                   jnp.where(alive, v, jnp.float32(jnp.inf)))
    rank = pos.astype(jnp.float32)
    mv, rank, i_f = _bitonic((mv, rank, i_f), W2, key2=True)
    od_ref[...] = mv[:, :KNN_OUT]
    oi_ref[...] = i_f[:, :KNN_OUT].astype(jnp.int32)


def _final_call(vals, idx, doc, qdoc):
    return pl.pallas_call(
        _final_body,
        grid=(NQ // QB,),
        in_specs=[
            pl.BlockSpec((QB, CAP), lambda i: (i, 0)),
            pl.BlockSpec((QB, CAP), lambda i: (i, 0)),
            pl.BlockSpec((QB, CAP), lambda i: (i, 0)),
            pl.BlockSpec((QB, 1), lambda i: (i, 0)),
        ],
        out_specs=[
            pl.BlockSpec((QB, KNN_OUT), lambda i: (i, 0)),
            pl.BlockSpec((QB, KNN_OUT), lambda i: (i, 0)),
        ],
        out_shape=[
            jax.ShapeDtypeStruct((NQ, KNN_OUT), jnp.float32),
            jax.ShapeDtypeStruct((NQ, KNN_OUT), jnp.int32),
        ],
    )(vals, idx, doc, qdoc)


def kernel(queries, keys, doc_ids, query_doc_ids, knn):
    keys_pad = jnp.pad(keys, ((0, N_PAD - N_KEYS), (0, 0)))
    q_sq = jnp.sum(queries * queries, axis=1, keepdims=True)
    k_sq = jnp.sum(keys_pad * keys_pad, axis=1)
    ks8 = jnp.broadcast_to(k_sq[None, :], (8, N_PAD))
    dist, mrep = _dist_call(queries, keys_pad, q_sq, ks8)
    tau = _tau_call(mrep)
    valsf, idxf, docf = _sc_select_call(
        dist.reshape(NQ * NCH, 128),
        mrep.reshape(NQ * NV),
        tau.reshape(NQ * 128),
        doc_ids,
    )
    dists_out, idx_out = _final_call(
        valsf.reshape(NQ, CAP),
        idxf.reshape(NQ, CAP),
        docf.reshape(NQ, CAP),
        query_doc_ids.reshape(NQ, 1),
    )
    return dists_out, idx_out


# speedup vs baseline: 1.4863x; 1.1581x over previous
import functools

import jax
import jax.numpy as jnp
from jax import lax
from jax.experimental import pallas as pl
from jax.experimental.pallas import tpu as pltpu
from jax.experimental.pallas import tpu_sc as plsc

KNN_EXTRA = 100
MASK_DISTANCE = 1000.0
KNN_OUT = 10
TOTAL_FETCH = KNN_OUT + KNN_EXTRA + 1

NQ = 1024
N_KEYS = 100000
N_PAD = 100352
NB = 2048
NV = N_PAD // 16
NCH = N_PAD // 128
QB = 256
CAP = 256
CHCAP = 512
GB = 128
RPW = 32


def _dist_body(q_ref, k_ref, qs_ref, ks_ref, d_ref, m16_ref):
    q = q_ref[...]
    k = k_ref[...]
    qk = lax.dot_general(q, k, (((1,), (1,)), ((), ())),
                         preferred_element_type=jnp.float32)
    d = qs_ref[...] - 2.0 * qk + ks_ref[...][0:1, :]
    j = pl.program_id(1)
    cols = j * NB + lax.broadcasted_iota(jnp.int32, (QB, NB), 1)
    d = jnp.where(cols < N_KEYS, d, jnp.float32(3e38))
    d_ref[...] = d.reshape(QB, NB // 128, 128)
    m = jnp.min(d.reshape(QB, NB // 128, 128), axis=-1)
    mrep = jnp.broadcast_to(m[:, :, None], (QB, NB // 128, 8))
    m16_ref[...] = mrep.reshape(QB, NB // 16)


def _dist_call(queries, keys_pad, q_sq, ks8):
    return pl.pallas_call(
        _dist_body,
        grid=(NQ // QB, N_PAD // NB),
        in_specs=[
            pl.BlockSpec((QB, 128), lambda i, j: (i, 0)),
            pl.BlockSpec((NB, 128), lambda i, j: (j, 0)),
            pl.BlockSpec((QB, 1), lambda i, j: (i, 0)),
            pl.BlockSpec((8, NB), lambda i, j: (0, j)),
        ],
        out_specs=[
            pl.BlockSpec((QB, NB // 128, 128), lambda i, j: (i, j, 0)),
            pl.BlockSpec((QB, NB // 16), lambda i, j: (i, j)),
        ],
        out_shape=[
            jax.ShapeDtypeStruct((NQ, NCH, 128), jnp.float32),
            jax.ShapeDtypeStruct((NQ, NV), jnp.float32),
        ],
        compiler_params=pltpu.CompilerParams(
            dimension_semantics=("arbitrary", "arbitrary"),
        ),
    )(queries, keys_pad, q_sq, ks8)


def _tau_body(mrep_ref, tau_ref):
    x = mrep_ref[...]
    lane = lax.broadcasted_iota(jnp.int32, (QB, NV), 1)
    rep_mask = (lane & 7) == 0
    inf = jnp.float32(jnp.inf)
    hm = jnp.where(rep_mask, x, inf)
    lo = jnp.min(x, axis=1, keepdims=True)
    real_mask = rep_mask & ((lane >> 3) <= (N_KEYS - 1) // 128)
    hi = jnp.max(jnp.where(real_mask, x, -inf), axis=1, keepdims=True)

    def count_le(t):
        return jnp.sum((hm <= t).astype(jnp.int32), axis=1, keepdims=True)

    def body(it, carry):
        lo, hi = carry
        mid = 0.5 * (lo + hi)
        ge = count_le(mid) >= TOTAL_FETCH
        hi = jnp.where(ge, mid, hi)
        lo = jnp.where(ge, lo, mid)
        return lo, hi

    lo, hi = lax.fori_loop(0, 26, body, (lo, hi))
    tau = jnp.where(count_le(lo) >= TOTAL_FETCH, lo, hi)
    tau_ref[...] = jnp.broadcast_to(tau, (QB, 128))


def _tau_call(mrep):
    return pl.pallas_call(
        _tau_body,
        grid=(NQ // QB,),
        in_specs=[pl.BlockSpec((QB, NV), lambda i: (i, 0))],
        out_specs=pl.BlockSpec((QB, 128), lambda i: (i, 0)),
        out_shape=jax.ShapeDtypeStruct((NQ, 128), jnp.float32),
    )(mrep)


def _sc_select_call(dist2dc, mrepf, tauf, doc_ids):
    mesh = plsc.VectorSubcoreMesh(core_axis_name="c", subcore_axis_name="s")
    nc = 2
    inf = jnp.float32(jnp.inf)

    @functools.partial(
        pl.kernel,
        out_type=(
            jax.ShapeDtypeStruct((NQ * CAP,), jnp.float32),
            jax.ShapeDtypeStruct((NQ * CAP,), jnp.int32),
            jax.ShapeDtypeStruct((NQ * CAP,), jnp.int32),
        ),
        mesh=mesh,
        scratch_types=[
            pltpu.VMEM((NV,), jnp.float32),
            pltpu.VMEM((RPW * 128,), jnp.float32),
            pltpu.VMEM((CHCAP,), jnp.int32),
            pltpu.VMEM((GB, 128), jnp.float32),
            pltpu.VMEM((CAP,), jnp.float32),
            pltpu.VMEM((CAP,), jnp.int32),
            pltpu.VMEM((CAP,), jnp.int32),
            pltpu.SemaphoreType.DMA,
        ],
        compiler_params=pltpu.CompilerParams(needs_layout_passes=False),
    )
    def sc_select(dist2dc, mrepf, tauf, doc_ids,
                  valsf, idxf, docf,
                  mrep_v, tau_v, cid_v, gbuf, cand_v, cand_i, cand_d, sem):
        wid = lax.axis_index("s") * nc + lax.axis_index("c")
        row0 = wid * RPW
        iota = lax.broadcasted_iota(jnp.int32, (16,), 0)
        pltpu.sync_copy(tauf.at[pl.ds(row0 * 128, RPW * 128)], tau_v)
        for z in range(CHCAP // 16):
            cid_v[pl.ds(z * 16, 16)] = iota + z * 16

        rep_mask = (iota & 7) == 0

        def row_body(r_local, _):
            row = row0 + r_local
            pltpu.sync_copy(mrepf.at[pl.ds(row * NV, NV)], mrep_v)
            tau_vec = tau_v[pl.ds(r_local * 128, 16)]

            def scan_body(g, cnt):
                m = mrep_v[pl.ds(g * 16, 16)]
                msk = jnp.logical_and(m <= tau_vec, rep_mask)
                key = jnp.where(msk, m, inf)
                ids = row * NCH + g * 2 + (iota >> 3)
                sk, sv = plsc.sort_key_val(key, ids)
                pos = jnp.minimum(cnt, CHCAP - 16)
                cid_v[pl.ds(pos, 16)] = sv
                npos = cnt + plsc.all_reduce_population_count(msk)[0]
                return jnp.minimum(npos, CHCAP - 16)

            cnt = lax.fori_loop(0, NV // 16, scan_body, jnp.int32(0))

            for z in range(CAP // 16):
                cand_v[pl.ds(z * 16, 16)] = jnp.full((16,), inf, jnp.float32)
                cand_i[pl.ds(z * 16, 16)] = iota + z * 16

            nb = (cnt + (GB - 1)) // GB

            def batch_body(b, ocnt):
                pltpu.async_copy(
                    dist2dc.at[cid_v.at[pl.ds(b * GB, GB)]], gbuf, sem
                ).wait()

                def grp_body(sg, ocnt):
                    cids16 = cid_v[pl.ds(b * GB + sg * 16, 16)]
                    for u in range(16):
                        gslot = sg * 16 + u
                        kb = (cids16[u] - row * NCH) * 128
                        valid = b * GB + gslot < cnt
                        for vv in range(8):
                            v = gbuf[gslot, pl.ds(vv * 16, 16)]
                            kidx = jnp.broadcast_to(kb + vv * 16, (16,)) + iota
                            keep = jnp.logical_and(
                                v <= tau_vec,
                                jnp.logical_and(valid, ocnt <= CAP - 16))
                            key = jnp.where(keep, v, inf)
                            sk, sv = plsc.sort_key_val(
                                key, jnp.where(keep, kidx, iota))
                            pos = jnp.minimum(ocnt, CAP - 16)
                            cand_v[pl.ds(pos, 16)] = sk
                            cand_i[pl.ds(pos, 16)] = sv
                            ocnt = ocnt + \
                                plsc.all_reduce_population_count(keep)[0]
                    return ocnt

                return lax.fori_loop(0, GB // 16, grp_body, ocnt)

            lax.fori_loop(0, nb, batch_body, jnp.int32(0))

            pltpu.async_copy(doc_ids.at[cand_i.at[pl.ds(0, 128)]],
                             cand_d.at[pl.ds(0, 128)], sem).wait()
            pltpu.async_copy(doc_ids.at[cand_i.at[pl.ds(128, 128)]],
                             cand_d.at[pl.ds(128, 128)], sem).wait()

            pltpu.sync_copy(cand_v, valsf.at[pl.ds(row * CAP, CAP)])
            pltpu.sync_copy(cand_i, idxf.at[pl.ds(row * CAP, CAP)])
            pltpu.sync_copy(cand_d, docf.at[pl.ds(row * CAP, CAP)])
            return 0

        lax.fori_loop(0, RPW, row_body, 0)

    return sc_select(dist2dc, mrepf, tauf, doc_ids)


def _lexlt(a_v, a_i, b_v, b_i):
    return (a_v < b_v) | ((a_v == b_v) & (a_i < b_i))


def _bitonic(arrs, width, key2):
    arrs = list(arrs)
    lane = lax.broadcasted_iota(jnp.int32, (QB, width), 1)
    k = 2
    while k <= width:
        j = k // 2
        while j >= 1:
            low = (lane & j) == 0
            partners = [jnp.where(low, jnp.roll(a, -j, axis=1),
                                  jnp.roll(a, j, axis=1)) for a in arrs]
            keep_min = low == ((lane & k) == 0)
            lt = _lexlt(arrs[0], arrs[1], partners[0], partners[1]) if key2 \
                else (arrs[0] < partners[0])
            take_self = lt == keep_min
            arrs = [jnp.where(take_self, a, p) for a, p in zip(arrs, partners)]
            j //= 2
        k *= 2
    return arrs


def _final_body(v_ref, i_ref, d_ref, qd_ref, od_ref, oi_ref):
    v = v_ref[...]
    i = i_ref[...]
    d = d_ref[...]
    qd = qd_ref[...]
    i_f = i.astype(jnp.float32)
    v, i_f, d_f = _bitonic((v, i_f, d.astype(jnp.float32)), CAP, key2=True)
    W2 = 128
    v, i_f, d_f = v[:, :W2], i_f[:, :W2], d_f[:, :W2]
    pos = lax.broadcasted_iota(jnp.int32, (QB, W2), 1)
    alive = pos < TOTAL_FETCH
    same = d_f == jnp.broadcast_to(qd.astype(jnp.float32), (QB, W2))
    mv = jnp.where(alive & same, jnp.float32(MASK_DISTANCE),
                   jnp.where(alive, v, jnp.float32(jnp.inf)))
    rank = pos.astype(jnp.float32)
    mv, rank, i_f = _bitonic((mv, rank, i_f), W2, key2=True)
    od_ref[...] = mv[:, :KNN_OUT]
    oi_ref[...] = i_f[:, :KNN_OUT].astype(jnp.int32)


def _final_call(vals, idx, doc, qdoc):
    return pl.pallas_call(
        _final_body,
        grid=(NQ // QB,),
        in_specs=[
            pl.BlockSpec((QB, CAP), lambda i: (i, 0)),
            pl.BlockSpec((QB, CAP), lambda i: (i, 0)),
            pl.BlockSpec((QB, CAP), lambda i: (i, 0)),
            pl.BlockSpec((QB, 1), lambda i: (i, 0)),
        ],
        out_specs=[
            pl.BlockSpec((QB, KNN_OUT), lambda i: (i, 0)),
            pl.BlockSpec((QB, KNN_OUT), lambda i: (i, 0)),
        ],
        out_shape=[
            jax.ShapeDtypeStruct((NQ, KNN_OUT), jnp.float32),
            jax.ShapeDtypeStruct((NQ, KNN_OUT), jnp.int32),
        ],
    )(vals, idx, doc, qdoc)


def kernel(queries, keys, doc_ids, query_doc_ids, knn):
    keys_pad = jnp.pad(keys, ((0, N_PAD - N_KEYS), (0, 0)))
    q_sq = jnp.sum(queries * queries, axis=1, keepdims=True)
    k_sq = jnp.sum(keys_pad * keys_pad, axis=1)
    ks8 = jnp.broadcast_to(k_sq[None, :], (8, N_PAD))
    dist, mrep = _dist_call(queries, keys_pad, q_sq, ks8)
    tau = _tau_call(mrep)
    valsf, idxf, docf = _sc_select_call(
        dist.reshape(NQ * NCH, 128),
        mrep.reshape(NQ * NV),
        tau.reshape(NQ * 128),
        doc_ids,
    )
    dists_out, idx_out = _final_call(
        valsf.reshape(NQ, CAP),
        idxf.reshape(NQ, CAP),
        docf.reshape(NQ, CAP),
        query_doc_ids.reshape(NQ, 1),
    )
    return dists_out, idx_out


# speedup vs baseline: 1.5005x; 1.0095x over previous
import functools

import jax
import jax.numpy as jnp
from jax import lax
from jax.experimental import pallas as pl
from jax.experimental.pallas import tpu as pltpu
from jax.experimental.pallas import tpu_sc as plsc

KNN_EXTRA = 100
MASK_DISTANCE = 1000.0
KNN_OUT = 10
TOTAL_FETCH = KNN_OUT + KNN_EXTRA + 1

NQ = 1024
N_KEYS = 100000
N_PAD = 100352
NB = 2048
NV = N_PAD // 16
NCH = N_PAD // 128
QB = 256
CAP = 256
CHCAP = 512
GB = 128
RPW = 32


def _dist_body(q_ref, k_ref, qs_ref, ks_ref, d_ref, m16_ref):
    q = q_ref[...]
    k = k_ref[...]
    qk = lax.dot_general(q, k, (((1,), (1,)), ((), ())),
                         preferred_element_type=jnp.float32)
    d = qs_ref[...] - 2.0 * qk + ks_ref[...][0:1, :]
    j = pl.program_id(1)
    cols = j * NB + lax.broadcasted_iota(jnp.int32, (QB, NB), 1)
    d = jnp.where(cols < N_KEYS, d, jnp.float32(3e38))
    d_ref[...] = d.reshape(QB, NB // 128, 128)
    m = jnp.min(d.reshape(QB, NB // 128, 128), axis=-1)
    mrep = jnp.broadcast_to(m[:, :, None], (QB, NB // 128, 8))
    m16_ref[...] = mrep.reshape(QB, NB // 16)


def _dist_call(queries, keys_pad, q_sq, ks8):
    return pl.pallas_call(
        _dist_body,
        grid=(NQ // QB, N_PAD // NB),
        in_specs=[
            pl.BlockSpec((QB, 128), lambda i, j: (i, 0)),
            pl.BlockSpec((NB, 128), lambda i, j: (j, 0)),
            pl.BlockSpec((QB, 1), lambda i, j: (i, 0)),
            pl.BlockSpec((8, NB), lambda i, j: (0, j)),
        ],
        out_specs=[
            pl.BlockSpec((QB, NB // 128, 128), lambda i, j: (i, j, 0)),
            pl.BlockSpec((QB, NB // 16), lambda i, j: (i, j)),
        ],
        out_shape=[
            jax.ShapeDtypeStruct((NQ, NCH, 128), jnp.float32),
            jax.ShapeDtypeStruct((NQ, NV), jnp.float32),
        ],
        compiler_params=pltpu.CompilerParams(
            dimension_semantics=("arbitrary", "arbitrary"),
        ),
    )(queries, keys_pad, q_sq, ks8)


def _tau_body(mrep_ref, tau_ref):
    x = mrep_ref[...]
    lane = lax.broadcasted_iota(jnp.int32, (QB, NV), 1)
    rep_mask = (lane & 7) == 0
    inf = jnp.float32(jnp.inf)
    hm = jnp.where(rep_mask, x, inf)
    lo = jnp.min(x, axis=1, keepdims=True)
    real_mask = rep_mask & ((lane >> 3) <= (N_KEYS - 1) // 128)
    hi = jnp.max(jnp.where(real_mask, x, -inf), axis=1, keepdims=True)

    def count_le(t):
        return jnp.sum((hm <= t).astype(jnp.int32), axis=1, keepdims=True)

    def body(it, carry):
        lo, hi = carry
        mid = 0.5 * (lo + hi)
        ge = count_le(mid) >= TOTAL_FETCH
        hi = jnp.where(ge, mid, hi)
        lo = jnp.where(ge, lo, mid)
        return lo, hi

    lo, hi = lax.fori_loop(0, 20, body, (lo, hi))
    tau = jnp.where(count_le(lo) >= TOTAL_FETCH, lo, hi)
    tau_ref[...] = jnp.broadcast_to(tau, (QB, 128))


def _tau_call(mrep):
    return pl.pallas_call(
        _tau_body,
        grid=(NQ // QB,),
        in_specs=[pl.BlockSpec((QB, NV), lambda i: (i, 0))],
        out_specs=pl.BlockSpec((QB, 128), lambda i: (i, 0)),
        out_shape=jax.ShapeDtypeStruct((NQ, 128), jnp.float32),
    )(mrep)


def _sc_select_call(dist2dc, mrepf, tauf, doc_ids):
    mesh = plsc.VectorSubcoreMesh(core_axis_name="c", subcore_axis_name="s")
    nc = 2
    inf = jnp.float32(jnp.inf)

    @functools.partial(
        pl.kernel,
        out_type=(
            jax.ShapeDtypeStruct((NQ * CAP,), jnp.float32),
            jax.ShapeDtypeStruct((NQ * CAP,), jnp.int32),
            jax.ShapeDtypeStruct((NQ * CAP,), jnp.int32),
        ),
        mesh=mesh,
        scratch_types=[
            pltpu.VMEM((NV,), jnp.float32),
            pltpu.VMEM((RPW * 128,), jnp.float32),
            pltpu.VMEM((CHCAP,), jnp.int32),
            pltpu.VMEM((GB, 128), jnp.float32),
            pltpu.VMEM((CAP,), jnp.float32),
            pltpu.VMEM((CAP,), jnp.int32),
            pltpu.VMEM((CAP,), jnp.int32),
            pltpu.SemaphoreType.DMA,
        ],
        compiler_params=pltpu.CompilerParams(needs_layout_passes=False),
    )
    def sc_select(dist2dc, mrepf, tauf, doc_ids,
                  valsf, idxf, docf,
                  mrep_v, tau_v, cid_v, gbuf, cand_v, cand_i, cand_d, sem):
        wid = lax.axis_index("s") * nc + lax.axis_index("c")
        row0 = wid * RPW
        iota = lax.broadcasted_iota(jnp.int32, (16,), 0)
        pltpu.sync_copy(tauf.at[pl.ds(row0 * 128, RPW * 128)], tau_v)
        for z in range(CHCAP // 16):
            cid_v[pl.ds(z * 16, 16)] = iota + z * 16

        rep_mask = (iota & 7) == 0

        def row_body(r_local, _):
            row = row0 + r_local
            pltpu.sync_copy(mrepf.at[pl.ds(row * NV, NV)], mrep_v)
            tau_vec = tau_v[pl.ds(r_local * 128, 16)]

            def scan_body(g, cnt):
                m = mrep_v[pl.ds(g * 16, 16)]
                msk = jnp.logical_and(m <= tau_vec, rep_mask)
                key = jnp.where(msk, m, inf)
                ids = row * NCH + g * 2 + (iota >> 3)
                sk, sv = plsc.sort_key_val(key, ids)
                pos = jnp.minimum(cnt, CHCAP - 16)
                cid_v[pl.ds(pos, 16)] = sv
                npos = cnt + plsc.all_reduce_population_count(msk)[0]
                return jnp.minimum(npos, CHCAP - 16)

            cnt = lax.fori_loop(0, NV // 16, scan_body, jnp.int32(0))

            for z in range(CAP // 16):
                cand_v[pl.ds(z * 16, 16)] = jnp.full((16,), inf, jnp.float32)
                cand_i[pl.ds(z * 16, 16)] = iota + z * 16

            nb = (cnt + (GB - 1)) // GB

            def batch_body(b, ocnt):
                pltpu.async_copy(
                    dist2dc.at[cid_v.at[pl.ds(b * GB, GB)]], gbuf, sem
                ).wait()

                def grp_body(sg, ocnt):
                    cids16 = cid_v[pl.ds(b * GB + sg * 16, 16)]
                    for u in range(16):
                        gslot = sg * 16 + u
                        kb = (cids16[u] - row * NCH) * 128
                        valid = b * GB + gslot < cnt
                        for vv in range(8):
                            v = gbuf[gslot, pl.ds(vv * 16, 16)]
                            kidx = jnp.broadcast_to(kb + vv * 16, (16,)) + iota
                            keep = jnp.logical_and(
                                v <= tau_vec,
                                jnp.logical_and(valid, ocnt <= CAP - 16))
                            key = jnp.where(keep, v, inf)
                            sk, sv = plsc.sort_key_val(
                                key, jnp.where(keep, kidx, iota))
                            pos = jnp.minimum(ocnt, CAP - 16)
                            cand_v[pl.ds(pos, 16)] = sk
                            cand_i[pl.ds(pos, 16)] = sv
                            ocnt = ocnt + \
                                plsc.all_reduce_population_count(keep)[0]
                    return ocnt

                return lax.fori_loop(0, GB // 16, grp_body, ocnt)

            lax.fori_loop(0, nb, batch_body, jnp.int32(0))

            pltpu.async_copy(doc_ids.at[cand_i.at[pl.ds(0, 128)]],
                             cand_d.at[pl.ds(0, 128)], sem).wait()
            pltpu.async_copy(doc_ids.at[cand_i.at[pl.ds(128, 128)]],
                             cand_d.at[pl.ds(128, 128)], sem).wait()

            pltpu.sync_copy(cand_v, valsf.at[pl.ds(row * CAP, CAP)])
            pltpu.sync_copy(cand_i, idxf.at[pl.ds(row * CAP, CAP)])
            pltpu.sync_copy(cand_d, docf.at[pl.ds(row * CAP, CAP)])
            return 0

        lax.fori_loop(0, RPW, row_body, 0)

    return sc_select(dist2dc, mrepf, tauf, doc_ids)


def _lexlt(a_v, a_i, b_v, b_i):
    return (a_v < b_v) | ((a_v == b_v) & (a_i < b_i))


def _bitonic(arrs, width, key2):
    arrs = list(arrs)
    lane = lax.broadcasted_iota(jnp.int32, (QB, width), 1)
    k = 2
    while k <= width:
        j = k // 2
        while j >= 1:
            low = (lane & j) == 0
            partners = [jnp.where(low, jnp.roll(a, -j, axis=1),
                                  jnp.roll(a, j, axis=1)) for a in arrs]
            keep_min = low == ((lane & k) == 0)
            lt = _lexlt(arrs[0], arrs[1], partners[0], partners[1]) if key2 \
                else (arrs[0] < partners[0])
            take_self = lt == keep_min
            arrs = [jnp.where(take_self, a, p) for a, p in zip(arrs, partners)]
            j //= 2
        k *= 2
    return arrs


def _final_body(v_ref, i_ref, d_ref, qd_ref, od_ref, oi_ref):
    v = v_ref[...]
    i = i_ref[...]
    d = d_ref[...]
    qd = qd_ref[...]
    i_f = i.astype(jnp.float32)
    v, i_f, d_f = _bitonic((v, i_f, d.astype(jnp.float32)), CAP, key2=True)
    W2 = 128
    v, i_f, d_f = v[:, :W2], i_f[:, :W2], d_f[:, :W2]
    pos = lax.broadcasted_iota(jnp.int32, (QB, W2), 1)
    alive = pos < TOTAL_FETCH
    same = d_f == jnp.broadcast_to(qd.astype(jnp.float32), (QB, W2))
    mv = jnp.where(alive & same, jnp.float32(MASK_DISTANCE),
                   jnp.where(alive, v, jnp.float32(jnp.inf)))
    rank = pos.astype(jnp.float32)
    mv, rank, i_f = _bitonic((mv, rank, i_f), W2, key2=True)
    od_ref[...] = mv[:, :KNN_OUT]
    oi_ref[...] = i_f[:, :KNN_OUT].astype(jnp.int32)


def _final_call(vals, idx, doc, qdoc):
    return pl.pallas_call(
        _final_body,
        grid=(NQ // QB,),
        in_specs=[
            pl.BlockSpec((QB, CAP), lambda i: (i, 0)),
            pl.BlockSpec((QB, CAP), lambda i: (i, 0)),
            pl.BlockSpec((QB, CAP), lambda i: (i, 0)),
            pl.BlockSpec((QB, 1), lambda i: (i, 0)),
        ],
        out_specs=[
            pl.BlockSpec((QB, KNN_OUT), lambda i: (i, 0)),
            pl.BlockSpec((QB, KNN_OUT), lambda i: (i, 0)),
        ],
        out_shape=[
            jax.ShapeDtypeStruct((NQ, KNN_OUT), jnp.float32),
            jax.ShapeDtypeStruct((NQ, KNN_OUT), jnp.int32),
        ],
    )(vals, idx, doc, qdoc)


def kernel(queries, keys, doc_ids, query_doc_ids, knn):
    keys_pad = jnp.pad(keys, ((0, N_PAD - N_KEYS), (0, 0)))
    q_sq = jnp.sum(queries * queries, axis=1, keepdims=True)
    k_sq = jnp.sum(keys_pad * keys_pad, axis=1)
    ks8 = jnp.broadcast_to(k_sq[None, :], (8, N_PAD))
    dist, mrep = _dist_call(queries, keys_pad, q_sq, ks8)
    tau = _tau_call(mrep)
    valsf, idxf, docf = _sc_select_call(
        dist.reshape(NQ * NCH, 128),
        mrep.reshape(NQ * NV),
        tau.reshape(NQ * 128),
        doc_ids,
    )
    dists_out, idx_out = _final_call(
        valsf.reshape(NQ, CAP),
        idxf.reshape(NQ, CAP),
        docf.reshape(NQ, CAP),
        query_doc_ids.reshape(NQ, 1),
    )
    return dists_out, idx_out
